# baseline (speedup 1.0000x reference)
import functools

import jax
import jax.numpy as jnp
from jax.experimental import pallas as pl
from jax.experimental.pallas import tpu as pltpu

_RB = 64
_CW = 2048


def _chunk_sums(lab, cl, s):
    m = lab == cl
    mf = jnp.where(m, 1.0, 0.0)
    t2 = jnp.maximum(s, 0.0)
    t1 = jnp.maximum(1.0 - s, 0.0)
    return (jnp.sum(t2), jnp.sum(t1 * mf), jnp.sum(t2 * mf), jnp.sum(mf))


def _loss_kernel(lab_ref, clab_ref, s_ref, out_ref, acc_ref, *, total, n):
    i = pl.program_id(0)
    nt = pl.num_programs(0)

    @pl.when(i == 0)
    def _init():
        acc_ref[0] = 0.0
        acc_ref[1] = 0.0
        acc_ref[2] = 0.0
        acc_ref[3] = 0.0

    lab = lab_ref[:]

    n_full = n // _CW

    def body(c, carry):
        s = s_ref[:, pl.ds(c * _CW, _CW)]
        cl = clab_ref[:, pl.ds(c * _CW, _CW)]
        d = _chunk_sums(lab, cl, s)
        return tuple(a + b for a, b in zip(carry, d))

    sums = jax.lax.fori_loop(0, n_full, body, (0.0, 0.0, 0.0, 0.0))

    tail = n - n_full * _CW
    if tail:
        d = _chunk_sums(
            lab,
            clab_ref[:, pl.ds(n_full * _CW, tail)],
            s_ref[:, pl.ds(n_full * _CW, tail)],
        )
        sums = tuple(a + b for a, b in zip(sums, d))

    acc_ref[0] += sums[0]
    acc_ref[1] += sums[1]
    acc_ref[2] += sums[2]
    acc_ref[3] += sums[3]

    @pl.when(i == nt - 1)
    def _fin():
        n_match = acc_ref[3]
        n_non = jnp.float32(total) - n_match
        out_ref[0] = acc_ref[1] / n_match + (acc_ref[0] - acc_ref[2]) / n_non


def kernel(fuse_scores, labels, center_labels):
    with jax.enable_x64(False):
        return _run(fuse_scores, labels, center_labels)


def _run(fuse_scores, labels, center_labels):
    B, N = fuse_scores.shape
    nt = B // _RB
    lab2d = labels.astype(jnp.int32).reshape(B, 1)
    clab2d = center_labels.astype(jnp.int32).reshape(1, N)

    out = pl.pallas_call(
        functools.partial(_loss_kernel, total=float(B) * float(N), n=N),
        grid=(nt,),
        in_specs=[
            pl.BlockSpec((_RB, 1), lambda i: (i, 0)),
            pl.BlockSpec((1, N), lambda i: (0, 0)),
            pl.BlockSpec((_RB, N), lambda i: (i, 0)),
        ],
        out_specs=pl.BlockSpec(memory_space=pltpu.SMEM),
        out_shape=jax.ShapeDtypeStruct((1,), jnp.float32),
        scratch_shapes=[
            pltpu.SMEM((4,), jnp.float32),
        ],
        compiler_params=pltpu.CompilerParams(
            vmem_limit_bytes=128 * 1024 * 1024,
        ),
    )(lab2d, clab2d, fuse_scores)
    score = out[0]
    return (score, score)


# speedup vs baseline: 1.1460x; 1.1460x over previous
import functools

import jax
import jax.numpy as jnp
from jax.experimental import pallas as pl
from jax.experimental.pallas import tpu as pltpu

_RB = 32


def _chunk_sums(lab, cl, s):
    m = lab == cl
    mf = jnp.where(m, 1.0, 0.0)
    t2 = jnp.maximum(s, 0.0)
    t1 = jnp.maximum(1.0 - s, 0.0)
    return (jnp.sum(t2), jnp.sum(t1 * mf), jnp.sum(t2 * mf), jnp.sum(mf))


def _loss_kernel(lab_ref, clab_ref, s_ref, out_ref, acc_ref, *, total, n):
    i = pl.program_id(0)
    nt = pl.num_programs(0)

    @pl.when(i == 0)
    def _init():
        acc_ref[0] = 0.0
        acc_ref[1] = 0.0
        acc_ref[2] = 0.0
        acc_ref[3] = 0.0

    lab = lab_ref[:]

    sums = _chunk_sums(lab, clab_ref[:], s_ref[:])
    acc_ref[0] += sums[0]
    acc_ref[1] += sums[1]
    acc_ref[2] += sums[2]
    acc_ref[3] += sums[3]

    @pl.when(i == nt - 1)
    def _fin():
        n_match = acc_ref[3]
        n_non = jnp.float32(total) - n_match
        out_ref[0] = acc_ref[1] / n_match + (acc_ref[0] - acc_ref[2]) / n_non


def kernel(fuse_scores, labels, center_labels):
    with jax.enable_x64(False):
        return _run(fuse_scores, labels, center_labels)


def _run(fuse_scores, labels, center_labels):
    B, N = fuse_scores.shape
    nt = B // _RB
    lab2d = labels.astype(jnp.int32).reshape(B, 1)
    clab2d = center_labels.astype(jnp.int32).reshape(1, N)

    out = pl.pallas_call(
        functools.partial(_loss_kernel, total=float(B) * float(N), n=N),
        grid=(nt,),
        in_specs=[
            pl.BlockSpec((_RB, 1), lambda i: (i, 0)),
            pl.BlockSpec((1, N), lambda i: (0, 0)),
            pl.BlockSpec((_RB, N), lambda i: (i, 0)),
        ],
        out_specs=pl.BlockSpec(memory_space=pltpu.SMEM),
        out_shape=jax.ShapeDtypeStruct((1,), jnp.float32),
        scratch_shapes=[
            pltpu.SMEM((4,), jnp.float32),
        ],
        compiler_params=pltpu.CompilerParams(
            vmem_limit_bytes=128 * 1024 * 1024,
        ),
    )(lab2d, clab2d, fuse_scores)
    score = out[0]
    return (score, score)
